# Initial kernel scaffold; baseline (speedup 1.0000x reference)
import jax
import jax.numpy as jnp
from jax.experimental import pallas as pl


def kernel(target_node_indice, x, edge_index, edge_time, node_time, edge_weight, W_self, W_nbr, b_dgn, gamma, beta, W2, b2, W_mu, b_mu, W_lv, b_lv):
    raise NotImplementedError("write your pallas kernel here")


# speedup vs baseline: 1.1171x; 1.1171x over previous
import jax
import jax.numpy as jnp
from jax.experimental import pallas as pl
from jax.experimental.pallas import tpu as pltpu

N = 10000
E = 320000
D = 128
B = 1024


def _dense_body(x_ref, agg_ref, ws_ref, wn_ref, b_ref, g_ref, be_ref,
                w2_ref, b2_ref, wmu_ref, bmu_ref, wlv_ref, blv_ref,
                mu_ref, lv_ref):
    x = x_ref[...]
    agg = agg_ref[...]
    h = x @ ws_ref[...] + agg @ wn_ref[...] + b_ref[...]
    h = jnp.maximum(h, 0.0)
    mean = jnp.mean(h, axis=0, keepdims=True)
    var = jnp.mean((h - mean) ** 2, axis=0, keepdims=True)
    h = (h - mean) * jax.lax.rsqrt(var + 1e-5) * g_ref[...] + be_ref[...]
    h = jnp.maximum(h, 0.0)
    h = jnp.maximum(h @ w2_ref[...] + b2_ref[...], 0.0)
    mu_ref[...] = h @ wmu_ref[...] + bmu_ref[...]
    lv_ref[...] = h @ wlv_ref[...] + blv_ref[...]


def _dense_stack(x, agg, W_self, W_nbr, b_dgn, gamma, beta, W2, b2,
                 W_mu, b_mu, W_lv, b_lv):
    r = lambda v: v.reshape(1, D)
    return pl.pallas_call(
        _dense_body,
        out_shape=(jax.ShapeDtypeStruct((N, D), jnp.float32),
                   jax.ShapeDtypeStruct((N, D), jnp.float32)),
    )(x, agg, W_self, W_nbr, r(b_dgn), r(gamma), r(beta), W2, r(b2),
      W_mu, r(b_mu), W_lv, r(b_lv))


def kernel(target_node_indice, x, edge_index, edge_time, node_time,
           edge_weight, W_self, W_nbr, b_dgn, gamma, beta, W2, b2,
           W_mu, b_mu, W_lv, b_lv):
    src = edge_index[0]
    dst = edge_index[1]
    tdiff = node_time[dst] - edge_time
    coeff = jnp.exp(-jnp.abs(tdiff)) * edge_weight
    msg = x[src] * coeff[:, None]
    agg = jax.ops.segment_sum(msg, dst, num_segments=N)

    mu_all, lv_all = _dense_stack(x, agg, W_self, W_nbr, b_dgn, gamma, beta,
                                  W2, b2, W_mu, b_mu, W_lv, b_lv)

    mu = mu_all[target_node_indice]
    log_var = lv_all[target_node_indice]
    std = jnp.exp(0.5 * log_var)
    eps = jax.random.normal(jax.random.key(1234), std.shape, dtype=std.dtype)
    z = mu + eps * std
    return (z, mu, log_var)


# speedup vs baseline: 10.0561x; 9.0023x over previous
import functools

import jax
import jax.numpy as jnp
from jax import lax
from jax.experimental import pallas as pl
from jax.experimental.pallas import tpu as pltpu
from jax.experimental.pallas import tpu_sc as plsc

N = 10000
E = 320000
D = 128
B = 1024

NC = 2
NS = 16
NW = NC * NS

EPW = E // NW
CH = 80
STG = 5
CPS = 25
EPS = CPS * CH
NPAD = 10240
RPW = NPAD // NS

_mesh = plsc.VectorSubcoreMesh(core_axis_name="c", subcore_axis_name="s")


def _agg_body(src_hbm, dst_hbm, et_hbm, ew_hbm, nt_hbm, x_hbm, zer_hbm,
              out_hbm, src_v, dst_v, et_v, ew_v, nt_v, rows_v, coeff_v,
              agg_sh, sem):
    cid = lax.axis_index("c")
    sid = lax.axis_index("s")
    wid = sid * NC + cid

    pltpu.sync_copy(nt_hbm, nt_v)
    pltpu.sync_copy(zer_hbm, agg_sh.at[pl.ds(sid * RPW, RPW)])
    plsc.subcore_barrier()

    def stage(stg, _):
        pltpu.sync_copy(src_hbm.at[wid, stg], src_v)
        pltpu.sync_copy(dst_hbm.at[wid, stg], dst_v)
        pltpu.sync_copy(et_hbm.at[wid, stg], et_v)
        pltpu.sync_copy(ew_hbm.at[wid, stg], ew_v)

        def chunk(q, _):
            base = pl.multiple_of(q * CH, CH)
            pltpu.async_copy(x_hbm.at[src_v.at[pl.ds(base, CH)]],
                             rows_v, sem).wait()
            for g in range(CH // 16):
                sl = pl.ds(g * 16, 16)
                ntv = plsc.load_gather(nt_v, [dst_v[q, sl]])
                cv = jnp.exp(-jnp.abs(ntv - et_v[pl.ds(base + g * 16, 16)]))
                coeff_v[sl] = cv * ew_v[pl.ds(base + g * 16, 16)]

            def row(r, _):
                cb = plsc.load_gather(coeff_v,
                                      [jnp.full((16,), r, jnp.int32)])
                for qq in range(D // 16):
                    sl2 = pl.ds(qq * 16, 16)
                    rows_v[r, sl2] = rows_v[r, sl2] * cb
                return _

            lax.fori_loop(0, CH, row, None)
            pltpu.sync_copy(rows_v, agg_sh.at[dst_v.at[q]], add=True)
            return _

        lax.fori_loop(0, CPS, chunk, None)
        return _

    lax.fori_loop(0, STG, stage, None)
    plsc.subcore_barrier()

    pltpu.sync_copy(agg_sh.at[pl.ds(sid * RPW, RPW)],
                    out_hbm.at[cid, pl.ds(sid * RPW, RPW)])


@functools.partial(
    pl.kernel,
    out_type=jax.ShapeDtypeStruct((NC, NPAD, D), jnp.float32),
    mesh=_mesh,
    compiler_params=pltpu.CompilerParams(needs_layout_passes=False),
    scratch_types=[
        pltpu.VMEM((EPS,), jnp.int32),
        pltpu.VMEM((CPS, CH), jnp.int32),
        pltpu.VMEM((EPS,), jnp.float32),
        pltpu.VMEM((EPS,), jnp.float32),
        pltpu.VMEM((N,), jnp.float32),
        pltpu.VMEM((CH, D), jnp.float32),
        pltpu.VMEM((CH,), jnp.float32),
        pltpu.VMEM_SHARED((NPAD, D), jnp.float32),
        pltpu.SemaphoreType.DMA,
    ],
)
def _sc_aggregate(src_hbm, dst_hbm, et_hbm, ew_hbm, nt_hbm, x_hbm, zer_hbm,
                  out_hbm, src_v, dst_v, et_v, ew_v, nt_v, rows_v, coeff_v,
                  agg_sh, sem):
    _agg_body(src_hbm, dst_hbm, et_hbm, ew_hbm, nt_hbm, x_hbm, zer_hbm,
              out_hbm, src_v, dst_v, et_v, ew_v, nt_v, rows_v, coeff_v,
              agg_sh, sem)


BPW = B // NW


def _head_body(idx_hbm, mu_hbm, lv_hbm, eps_hbm, z_out, mu_out, lv_out,
               idx_v, mu_v, lv_v, eps_v, z_v, sem1, sem2):
    cid = lax.axis_index("c")
    sid = lax.axis_index("s")
    wid = sid * NC + cid
    base = wid * BPW
    pltpu.sync_copy(idx_hbm.at[pl.ds(base, BPW)], idx_v)
    cp1 = pltpu.async_copy(mu_hbm.at[idx_v], mu_v, sem1)
    cp2 = pltpu.async_copy(lv_hbm.at[idx_v], lv_v, sem2)
    pltpu.sync_copy(eps_hbm.at[pl.ds(base, BPW)], eps_v)
    cp1.wait()
    cp2.wait()

    def row(r, _):
        for q in range(D // 16):
            sl = pl.ds(q * 16, 16)
            std = jnp.exp(0.5 * lv_v[r, sl])
            z_v[r, sl] = mu_v[r, sl] + eps_v[r, sl] * std
        return _

    lax.fori_loop(0, BPW, row, None)
    pltpu.sync_copy(z_v, z_out.at[pl.ds(base, BPW)])
    pltpu.sync_copy(mu_v, mu_out.at[pl.ds(base, BPW)])
    pltpu.sync_copy(lv_v, lv_out.at[pl.ds(base, BPW)])


@functools.partial(
    pl.kernel,
    out_type=(jax.ShapeDtypeStruct((B, D), jnp.float32),
              jax.ShapeDtypeStruct((B, D), jnp.float32),
              jax.ShapeDtypeStruct((B, D), jnp.float32)),
    mesh=_mesh,
    compiler_params=pltpu.CompilerParams(needs_layout_passes=False),
    scratch_types=[
        pltpu.VMEM((BPW,), jnp.int32),
        pltpu.VMEM((BPW, D), jnp.float32),
        pltpu.VMEM((BPW, D), jnp.float32),
        pltpu.VMEM((BPW, D), jnp.float32),
        pltpu.VMEM((BPW, D), jnp.float32),
        pltpu.SemaphoreType.DMA,
        pltpu.SemaphoreType.DMA,
    ],
)
def _sc_head(idx_hbm, mu_hbm, lv_hbm, eps_hbm, z_out, mu_out, lv_out,
             idx_v, mu_v, lv_v, eps_v, z_v, sem1, sem2):
    _head_body(idx_hbm, mu_hbm, lv_hbm, eps_hbm, z_out, mu_out, lv_out,
               idx_v, mu_v, lv_v, eps_v, z_v, sem1, sem2)


def _dense_body(x_ref, agg_ref, ws_ref, wn_ref, b_ref, g_ref, be_ref,
                w2_ref, b2_ref, wmu_ref, bmu_ref, wlv_ref, blv_ref,
                mu_ref, lv_ref):
    x = x_ref[...]
    agg = agg_ref[0, :N] + agg_ref[1, :N]
    h = x @ ws_ref[...] + agg @ wn_ref[...] + b_ref[...]
    h = jnp.maximum(h, 0.0)
    mean = jnp.mean(h, axis=0, keepdims=True)
    var = jnp.mean((h - mean) ** 2, axis=0, keepdims=True)
    h = (h - mean) * lax.rsqrt(var + 1e-5) * g_ref[...] + be_ref[...]
    h = jnp.maximum(h, 0.0)
    h = jnp.maximum(h @ w2_ref[...] + b2_ref[...], 0.0)
    mu_ref[...] = h @ wmu_ref[...] + bmu_ref[...]
    lv_ref[...] = h @ wlv_ref[...] + blv_ref[...]


def _dense_stack(x, agg_parts, W_self, W_nbr, b_dgn, gamma, beta, W2, b2,
                 W_mu, b_mu, W_lv, b_lv):
    r = lambda v: v.reshape(1, D)
    return pl.pallas_call(
        _dense_body,
        out_shape=(jax.ShapeDtypeStruct((N, D), jnp.float32),
                   jax.ShapeDtypeStruct((N, D), jnp.float32)),
    )(x, agg_parts, W_self, W_nbr, r(b_dgn), r(gamma), r(beta), W2, r(b2),
      W_mu, r(b_mu), W_lv, r(b_lv))


def kernel(target_node_indice, x, edge_index, edge_time, node_time,
           edge_weight, W_self, W_nbr, b_dgn, gamma, beta, W2, b2,
           W_mu, b_mu, W_lv, b_lv):
    src = edge_index[0].astype(jnp.int32).reshape(NW, STG, EPS)
    dst = edge_index[1].astype(jnp.int32).reshape(NW, STG, CPS, CH)
    et = edge_time.reshape(NW, STG, EPS)
    ew = edge_weight.reshape(NW, STG, EPS)
    zer = jnp.zeros((RPW, D), jnp.float32)

    agg_parts = _sc_aggregate(src, dst, et, ew, node_time, x, zer)

    mu_all, lv_all = _dense_stack(x, agg_parts, W_self, W_nbr, b_dgn, gamma,
                                  beta, W2, b2, W_mu, b_mu, W_lv, b_lv)

    eps = jax.random.normal(jax.random.key(1234), (B, D), dtype=jnp.float32)
    idx = target_node_indice.astype(jnp.int32)
    z, mu, log_var = _sc_head(idx, mu_all, lv_all, eps)
    return (z, mu, log_var)


# speedup vs baseline: 17.4254x; 1.7328x over previous
import functools

import jax
import jax.numpy as jnp
from jax import lax
from jax.experimental import pallas as pl
from jax.experimental.pallas import tpu as pltpu
from jax.experimental.pallas import tpu_sc as plsc

N = 10000
E = 320000
D = 128
B = 1024

NC = 2
NS = 16
NW = NC * NS

EPW = E // NW
CH = 80
STG = 5
CPS = 25
EPS = CPS * CH
NPAD = 10240
RPW = NPAD // NS

_mesh = plsc.VectorSubcoreMesh(core_axis_name="c", subcore_axis_name="s")


def _agg_body(src_hbm, dst_hbm, et_hbm, ew_hbm, nt_hbm, x_hbm, zer_hbm,
              out_hbm, src_v, dst_v, et_v, ew_v, ntg_v, rows_v, coeff_v,
              agg_sh, *sems):
    grsem = sems[0:3]
    gnsem = sems[3:6]
    ssem = sems[6:9]
    cid = lax.axis_index("c")
    sid = lax.axis_index("s")
    wid = sid * NC + cid

    pltpu.sync_copy(zer_hbm, agg_sh.at[pl.ds(sid * RPW, RPW)])
    plsc.subcore_barrier()

    def stage(stg, _):
        pltpu.sync_copy(src_hbm.at[wid, stg], src_v)
        pltpu.sync_copy(dst_hbm.at[wid, stg], dst_v)
        pltpu.sync_copy(et_hbm.at[wid, stg], et_v)
        pltpu.sync_copy(ew_hbm.at[wid, stg], ew_v)

        g_desc = {}
        s_desc = {}

        def issue_g(c):
            p = c % 3
            d1 = pltpu.make_async_copy(
                x_hbm.at[src_v.at[pl.ds(c * CH, CH)]], rows_v.at[p],
                grsem[p])
            d1.start()
            d2 = pltpu.make_async_copy(nt_hbm.at[dst_v.at[c]], ntg_v.at[p],
                                       gnsem[p])
            d2.start()
            g_desc[c] = (d1, d2)

        issue_g(0)
        issue_g(1)
        for c in range(CPS):
            p = c % 3
            d1, d2 = g_desc.pop(c)
            d1.wait()
            d2.wait()
            for g in range(CH // 16):
                sl = pl.ds(g * 16, 16)
                cv = jnp.exp(-jnp.abs(ntg_v[p, sl]
                                      - et_v[pl.ds(c * CH + g * 16, 16)]))
                coeff_v[sl] = cv * ew_v[pl.ds(c * CH + g * 16, 16)]

            def row(r, _, p=p):
                cb = plsc.load_gather(coeff_v,
                                      [jnp.full((16,), r, jnp.int32)])
                for qq in range(D // 16):
                    sl2 = pl.ds(qq * 16, 16)
                    rows_v[p, r, sl2] = rows_v[p, r, sl2] * cb
                return _

            lax.fori_loop(0, CH, row, None)
            sd = pltpu.make_async_copy(rows_v.at[p], agg_sh.at[dst_v.at[c]],
                                       ssem[p])
            sd.start(add=True)
            s_desc[c] = sd
            if c + 2 < CPS:
                if c >= 1:
                    s_desc.pop(c - 1).wait()
                issue_g(c + 2)
        for c in sorted(s_desc):
            s_desc.pop(c).wait()
        return _

    lax.fori_loop(0, STG, stage, None)
    plsc.subcore_barrier()

    pltpu.sync_copy(agg_sh.at[pl.ds(sid * RPW, RPW)],
                    out_hbm.at[cid, pl.ds(sid * RPW, RPW)])


@functools.partial(
    pl.kernel,
    out_type=jax.ShapeDtypeStruct((NC, NPAD, D), jnp.float32),
    mesh=_mesh,
    compiler_params=pltpu.CompilerParams(needs_layout_passes=False),
    scratch_types=[
        pltpu.VMEM((EPS,), jnp.int32),
        pltpu.VMEM((CPS, CH), jnp.int32),
        pltpu.VMEM((EPS,), jnp.float32),
        pltpu.VMEM((EPS,), jnp.float32),
        pltpu.VMEM((3, CH), jnp.float32),
        pltpu.VMEM((3, CH, D), jnp.float32),
        pltpu.VMEM((CH,), jnp.float32),
        pltpu.VMEM_SHARED((NPAD, D), jnp.float32),
        pltpu.SemaphoreType.DMA,
        pltpu.SemaphoreType.DMA,
        pltpu.SemaphoreType.DMA,
        pltpu.SemaphoreType.DMA,
        pltpu.SemaphoreType.DMA,
        pltpu.SemaphoreType.DMA,
        pltpu.SemaphoreType.DMA,
        pltpu.SemaphoreType.DMA,
        pltpu.SemaphoreType.DMA,
    ],
)
def _sc_aggregate(src_hbm, dst_hbm, et_hbm, ew_hbm, nt_hbm, x_hbm, zer_hbm,
                  out_hbm, src_v, dst_v, et_v, ew_v, ntg_v, rows_v, coeff_v,
                  agg_sh, *sems):
    _agg_body(src_hbm, dst_hbm, et_hbm, ew_hbm, nt_hbm, x_hbm, zer_hbm,
              out_hbm, src_v, dst_v, et_v, ew_v, ntg_v, rows_v, coeff_v,
              agg_sh, *sems)


BPW = B // NW


def _head_body(idx_hbm, mu_hbm, lv_hbm, eps_hbm, z_out, mu_out, lv_out,
               idx_v, mu_v, lv_v, eps_v, z_v, sem1, sem2):
    cid = lax.axis_index("c")
    sid = lax.axis_index("s")
    wid = sid * NC + cid
    base = wid * BPW
    pltpu.sync_copy(idx_hbm.at[pl.ds(base, BPW)], idx_v)
    cp1 = pltpu.async_copy(mu_hbm.at[idx_v], mu_v, sem1)
    cp2 = pltpu.async_copy(lv_hbm.at[idx_v], lv_v, sem2)
    pltpu.sync_copy(eps_hbm.at[pl.ds(base, BPW)], eps_v)
    cp1.wait()
    cp2.wait()

    def row(r, _):
        for q in range(D // 16):
            sl = pl.ds(q * 16, 16)
            std = jnp.exp(0.5 * lv_v[r, sl])
            z_v[r, sl] = mu_v[r, sl] + eps_v[r, sl] * std
        return _

    lax.fori_loop(0, BPW, row, None)
    pltpu.sync_copy(z_v, z_out.at[pl.ds(base, BPW)])
    pltpu.sync_copy(mu_v, mu_out.at[pl.ds(base, BPW)])
    pltpu.sync_copy(lv_v, lv_out.at[pl.ds(base, BPW)])


@functools.partial(
    pl.kernel,
    out_type=(jax.ShapeDtypeStruct((B, D), jnp.float32),
              jax.ShapeDtypeStruct((B, D), jnp.float32),
              jax.ShapeDtypeStruct((B, D), jnp.float32)),
    mesh=_mesh,
    compiler_params=pltpu.CompilerParams(needs_layout_passes=False),
    scratch_types=[
        pltpu.VMEM((BPW,), jnp.int32),
        pltpu.VMEM((BPW, D), jnp.float32),
        pltpu.VMEM((BPW, D), jnp.float32),
        pltpu.VMEM((BPW, D), jnp.float32),
        pltpu.VMEM((BPW, D), jnp.float32),
        pltpu.SemaphoreType.DMA,
        pltpu.SemaphoreType.DMA,
    ],
)
def _sc_head(idx_hbm, mu_hbm, lv_hbm, eps_hbm, z_out, mu_out, lv_out,
             idx_v, mu_v, lv_v, eps_v, z_v, sem1, sem2):
    _head_body(idx_hbm, mu_hbm, lv_hbm, eps_hbm, z_out, mu_out, lv_out,
               idx_v, mu_v, lv_v, eps_v, z_v, sem1, sem2)


def _dense_body(x_ref, agg_ref, ws_ref, wn_ref, b_ref, g_ref, be_ref,
                w2_ref, b2_ref, wmu_ref, bmu_ref, wlv_ref, blv_ref,
                mu_ref, lv_ref):
    x = x_ref[...]
    agg = agg_ref[0, :N] + agg_ref[1, :N]
    h = x @ ws_ref[...] + agg @ wn_ref[...] + b_ref[...]
    h = jnp.maximum(h, 0.0)
    mean = jnp.mean(h, axis=0, keepdims=True)
    var = jnp.mean((h - mean) ** 2, axis=0, keepdims=True)
    h = (h - mean) * lax.rsqrt(var + 1e-5) * g_ref[...] + be_ref[...]
    h = jnp.maximum(h, 0.0)
    h = jnp.maximum(h @ w2_ref[...] + b2_ref[...], 0.0)
    mu_ref[...] = h @ wmu_ref[...] + bmu_ref[...]
    lv_ref[...] = h @ wlv_ref[...] + blv_ref[...]


def _dense_stack(x, agg_parts, W_self, W_nbr, b_dgn, gamma, beta, W2, b2,
                 W_mu, b_mu, W_lv, b_lv):
    r = lambda v: v.reshape(1, D)
    return pl.pallas_call(
        _dense_body,
        out_shape=(jax.ShapeDtypeStruct((N, D), jnp.float32),
                   jax.ShapeDtypeStruct((N, D), jnp.float32)),
    )(x, agg_parts, W_self, W_nbr, r(b_dgn), r(gamma), r(beta), W2, r(b2),
      W_mu, r(b_mu), W_lv, r(b_lv))


def kernel(target_node_indice, x, edge_index, edge_time, node_time,
           edge_weight, W_self, W_nbr, b_dgn, gamma, beta, W2, b2,
           W_mu, b_mu, W_lv, b_lv):
    src = edge_index[0].astype(jnp.int32).reshape(NW, STG, EPS)
    dst = edge_index[1].astype(jnp.int32).reshape(NW, STG, CPS, CH)
    et = edge_time.reshape(NW, STG, EPS)
    ew = edge_weight.reshape(NW, STG, EPS)
    zer = jnp.zeros((RPW, D), jnp.float32)

    agg_parts = _sc_aggregate(src, dst, et, ew, node_time, x, zer)

    mu_all, lv_all = _dense_stack(x, agg_parts, W_self, W_nbr, b_dgn, gamma,
                                  beta, W2, b2, W_mu, b_mu, W_lv, b_lv)

    eps = jax.random.normal(jax.random.key(1234), (B, D), dtype=jnp.float32)
    idx = target_node_indice.astype(jnp.int32)
    z, mu, log_var = _sc_head(idx, mu_all, lv_all, eps)
    return (z, mu, log_var)


# speedup vs baseline: 21.2772x; 1.2210x over previous
import functools

import jax
import jax.numpy as jnp
from jax import lax
from jax.experimental import pallas as pl
from jax.experimental.pallas import tpu as pltpu
from jax.experimental.pallas import tpu_sc as plsc

N = 10000
E = 320000
D = 128
B = 1024

NC = 2
NS = 16
NW = NC * NS

EPW = E // NW
CH = 80
STG = 5
CPS = 25
EPS = CPS * CH
NPAD = 10240
RPW = NPAD // NS

_mesh = plsc.VectorSubcoreMesh(core_axis_name="c", subcore_axis_name="s")


def _agg_body(ei_hbm, et_hbm, ew_hbm, nt_hbm, x_hbm, zer_hbm,
              out_hbm, src_v, dst_v, et_v, ew_v, ntg_v, rows_v, coeff_v,
              agg_sh, nt_sh, *sems):
    grsem = sems[0:3]
    gnsem = sems[3:6]
    ssem = sems[6:9]
    cid = lax.axis_index("c")
    sid = lax.axis_index("s")
    wid = sid * NC + cid

    pltpu.sync_copy(zer_hbm, agg_sh.at[pl.ds(sid * RPW, RPW)])

    @pl.when(sid == 0)
    def _():
        pltpu.sync_copy(nt_hbm, nt_sh)

    plsc.subcore_barrier()

    def stage(stg, _):
        base = pl.multiple_of(wid * EPW + stg * EPS, EPS)
        stcps = [
            pltpu.make_async_copy(ei_hbm.at[pl.ds(base, EPS)], src_v,
                                  grsem[0]),
            pltpu.make_async_copy(ei_hbm.at[pl.ds(E + base, EPS)], dst_v,
                                  grsem[1]),
            pltpu.make_async_copy(et_hbm.at[pl.ds(base, EPS)], et_v,
                                  grsem[2]),
            pltpu.make_async_copy(ew_hbm.at[pl.ds(base, EPS)], ew_v,
                                  gnsem[0]),
        ]
        for cp in stcps:
            cp.start()
        for cp in stcps:
            cp.wait()

        g_desc = {}
        s_desc = {}

        def issue_g(c):
            p = c % 3
            d1 = pltpu.make_async_copy(
                x_hbm.at[src_v.at[pl.ds(c * CH, CH)]], rows_v.at[p],
                grsem[p])
            d1.start()
            d2 = pltpu.make_async_copy(nt_sh.at[dst_v.at[pl.ds(c * CH, CH)]],
                                       ntg_v.at[p], gnsem[p])
            d2.start()
            g_desc[c] = (d1, d2)

        issue_g(0)
        issue_g(1)
        for c in range(CPS):
            p = c % 3
            d1, d2 = g_desc.pop(c)
            d1.wait()
            d2.wait()
            for g in range(CH // 16):
                sl = pl.ds(g * 16, 16)
                cv = jnp.exp(-jnp.abs(ntg_v[p, sl]
                                      - et_v[pl.ds(c * CH + g * 16, 16)]))
                coeff_v[sl] = cv * ew_v[pl.ds(c * CH + g * 16, 16)]

            def row(r8, _, p=p):
                r0 = pl.multiple_of(r8 * 8, 8)
                cbase = jnp.full((16,), r0, jnp.int32)
                cbs = [plsc.load_gather(coeff_v, [cbase + dr])
                       for dr in range(8)]
                for dr in range(8):
                    for qq in range(D // 16):
                        sl2 = pl.ds(qq * 16, 16)
                        rows_v[p, r0 + dr, sl2] = (rows_v[p, r0 + dr, sl2]
                                                   * cbs[dr])
                return _

            lax.fori_loop(0, CH // 8, row, None)
            sd = pltpu.make_async_copy(
                rows_v.at[p], agg_sh.at[dst_v.at[pl.ds(c * CH, CH)]],
                ssem[p])
            sd.start(add=True)
            s_desc[c] = sd
            if c + 2 < CPS:
                if c >= 1:
                    s_desc.pop(c - 1).wait()
                issue_g(c + 2)
        for c in sorted(s_desc):
            s_desc.pop(c).wait()
        return _

    lax.fori_loop(0, STG, stage, None)
    plsc.subcore_barrier()

    pltpu.sync_copy(agg_sh.at[pl.ds(sid * RPW, RPW)],
                    out_hbm.at[cid, pl.ds(sid * RPW, RPW)])


@functools.partial(
    pl.kernel,
    out_type=jax.ShapeDtypeStruct((NC, NPAD, D), jnp.float32),
    mesh=_mesh,
    compiler_params=pltpu.CompilerParams(needs_layout_passes=False),
    scratch_types=[
        pltpu.VMEM((EPS,), jnp.int32),
        pltpu.VMEM((EPS,), jnp.int32),
        pltpu.VMEM((EPS,), jnp.float32),
        pltpu.VMEM((EPS,), jnp.float32),
        pltpu.VMEM((3, CH), jnp.float32),
        pltpu.VMEM((3, CH, D), jnp.float32),
        pltpu.VMEM((CH,), jnp.float32),
        pltpu.VMEM_SHARED((NPAD, D), jnp.float32),
        pltpu.VMEM_SHARED((N,), jnp.float32),
        pltpu.SemaphoreType.DMA,
        pltpu.SemaphoreType.DMA,
        pltpu.SemaphoreType.DMA,
        pltpu.SemaphoreType.DMA,
        pltpu.SemaphoreType.DMA,
        pltpu.SemaphoreType.DMA,
        pltpu.SemaphoreType.DMA,
        pltpu.SemaphoreType.DMA,
        pltpu.SemaphoreType.DMA,
    ],
)
def _sc_aggregate(ei_hbm, et_hbm, ew_hbm, nt_hbm, x_hbm, zer_hbm,
                  out_hbm, src_v, dst_v, et_v, ew_v, ntg_v, rows_v, coeff_v,
                  agg_sh, nt_sh, *sems):
    _agg_body(ei_hbm, et_hbm, ew_hbm, nt_hbm, x_hbm, zer_hbm,
              out_hbm, src_v, dst_v, et_v, ew_v, ntg_v, rows_v, coeff_v,
              agg_sh, nt_sh, *sems)


BPW = B // NW


def _head_body(idx_hbm, mu_hbm, lv_hbm, eps_hbm, z_out, mu_out, lv_out,
               idx_v, mu_v, lv_v, eps_v, z_v, sem1, sem2):
    cid = lax.axis_index("c")
    sid = lax.axis_index("s")
    wid = sid * NC + cid
    base = wid * BPW
    pltpu.sync_copy(idx_hbm.at[pl.ds(base, BPW)], idx_v)
    cp1 = pltpu.async_copy(mu_hbm.at[idx_v], mu_v, sem1)
    cp2 = pltpu.async_copy(lv_hbm.at[idx_v], lv_v, sem2)
    pltpu.sync_copy(eps_hbm.at[pl.ds(base, BPW)], eps_v)
    cp1.wait()
    cp2.wait()

    def row(r, _):
        for q in range(D // 16):
            sl = pl.ds(q * 16, 16)
            std = jnp.exp(0.5 * lv_v[r, sl])
            z_v[r, sl] = mu_v[r, sl] + eps_v[r, sl] * std
        return _

    lax.fori_loop(0, BPW, row, None)
    pltpu.sync_copy(z_v, z_out.at[pl.ds(base, BPW)])
    pltpu.sync_copy(mu_v, mu_out.at[pl.ds(base, BPW)])
    pltpu.sync_copy(lv_v, lv_out.at[pl.ds(base, BPW)])


@functools.partial(
    pl.kernel,
    out_type=(jax.ShapeDtypeStruct((B, D), jnp.float32),
              jax.ShapeDtypeStruct((B, D), jnp.float32),
              jax.ShapeDtypeStruct((B, D), jnp.float32)),
    mesh=_mesh,
    compiler_params=pltpu.CompilerParams(needs_layout_passes=False),
    scratch_types=[
        pltpu.VMEM((BPW,), jnp.int32),
        pltpu.VMEM((BPW, D), jnp.float32),
        pltpu.VMEM((BPW, D), jnp.float32),
        pltpu.VMEM((BPW, D), jnp.float32),
        pltpu.VMEM((BPW, D), jnp.float32),
        pltpu.SemaphoreType.DMA,
        pltpu.SemaphoreType.DMA,
    ],
)
def _sc_head(idx_hbm, mu_hbm, lv_hbm, eps_hbm, z_out, mu_out, lv_out,
             idx_v, mu_v, lv_v, eps_v, z_v, sem1, sem2):
    _head_body(idx_hbm, mu_hbm, lv_hbm, eps_hbm, z_out, mu_out, lv_out,
               idx_v, mu_v, lv_v, eps_v, z_v, sem1, sem2)


def _dense_body(x_ref, agg_ref, ws_ref, wn_ref, b_ref, g_ref, be_ref,
                w2_ref, b2_ref, wmu_ref, bmu_ref, wlv_ref, blv_ref,
                mu_ref, lv_ref):
    x = x_ref[...]
    agg = agg_ref[0, :N] + agg_ref[1, :N]
    h = x @ ws_ref[...] + agg @ wn_ref[...] + b_ref[...]
    h = jnp.maximum(h, 0.0)
    mean = jnp.mean(h, axis=0, keepdims=True)
    var = jnp.mean((h - mean) ** 2, axis=0, keepdims=True)
    h = (h - mean) * lax.rsqrt(var + 1e-5) * g_ref[...] + be_ref[...]
    h = jnp.maximum(h, 0.0)
    h = jnp.maximum(h @ w2_ref[...] + b2_ref[...], 0.0)
    mu_ref[...] = h @ wmu_ref[...] + bmu_ref[...]
    lv_ref[...] = h @ wlv_ref[...] + blv_ref[...]


def _dense_stack(x, agg_parts, W_self, W_nbr, b_dgn, gamma, beta, W2, b2,
                 W_mu, b_mu, W_lv, b_lv):
    r = lambda v: v.reshape(1, D)
    return pl.pallas_call(
        _dense_body,
        out_shape=(jax.ShapeDtypeStruct((N, D), jnp.float32),
                   jax.ShapeDtypeStruct((N, D), jnp.float32)),
    )(x, agg_parts, W_self, W_nbr, r(b_dgn), r(gamma), r(beta), W2, r(b2),
      W_mu, r(b_mu), W_lv, r(b_lv))


def kernel(target_node_indice, x, edge_index, edge_time, node_time,
           edge_weight, W_self, W_nbr, b_dgn, gamma, beta, W2, b2,
           W_mu, b_mu, W_lv, b_lv):
    ei = edge_index.astype(jnp.int32).reshape(2 * E)
    zer = jnp.zeros((RPW, D), jnp.float32)

    agg_parts = _sc_aggregate(ei, edge_time, edge_weight, node_time, x, zer)

    mu_all, lv_all = _dense_stack(x, agg_parts, W_self, W_nbr, b_dgn, gamma,
                                  beta, W2, b2, W_mu, b_mu, W_lv, b_lv)

    eps = jax.random.normal(jax.random.key(1234), (B, D), dtype=jnp.float32)
    idx = target_node_indice.astype(jnp.int32)
    z, mu, log_var = _sc_head(idx, mu_all, lv_all, eps)
    return (z, mu, log_var)
